# baseline (speedup 1.0000x reference)
import functools

import jax
import jax.numpy as jnp
from jax import lax
from jax.experimental import pallas as pl
from jax.experimental.pallas import tpu as pltpu
from jax.experimental.pallas import tpu_sc as plsc

_NBUF = 8
_CHUNK = 64
_LANES = 16


def _make_sc_gather(n, c, m):
  info = plsc.get_sparse_core_info()
  nw = info.num_cores * info.num_subcores
  rows_per_w = m // nw
  n_chunks = rows_per_w // _CHUNK
  n_groups = n_chunks // _NBUF
  assert m == nw * rows_per_w and rows_per_w == n_chunks * _CHUNK
  assert n_chunks == n_groups * _NBUF

  mesh = plsc.VectorSubcoreMesh(core_axis_name="c", subcore_axis_name="s")

  @functools.partial(
      pl.kernel,
      out_type=jax.ShapeDtypeStruct((m, c), jnp.float32),
      mesh=mesh,
      scratch_types=(
          [pltpu.VMEM((rows_per_w,), jnp.int32)]
          + [pltpu.VMEM((_CHUNK, c), jnp.float32) for _ in range(_NBUF)]
          + [pltpu.SemaphoreType.DMA for _ in range(_NBUF)]
      ),
  )
  def gather_kernel(data_hbm, idx_hbm, out_hbm, idx_v, *bufs_sems):
    bufs = bufs_sems[:_NBUF]
    sems = bufs_sems[_NBUF:]
    wid = lax.axis_index("s") * info.num_cores + lax.axis_index("c")
    base = wid * rows_per_w

    pltpu.sync_copy(idx_hbm.at[pl.ds(base, rows_per_w)], idx_v)

    def shift_chunk(chunk):
      for i in range(_CHUNK // _LANES):
        sl = pl.ds(chunk * _CHUNK + i * _LANES, _LANES)
        idx_v[sl] = lax.shift_right_logical(idx_v[sl], 3)

    def start(chunk, b):
      pltpu.async_copy(
          data_hbm.at[idx_v.at[pl.ds(chunk * _CHUNK, _CHUNK)]],
          bufs[b],
          sems[b],
      )

    def drain(chunk, b):
      pltpu.make_async_copy(
          data_hbm.at[idx_v.at[pl.ds(chunk * _CHUNK, _CHUNK)]],
          bufs[b],
          sems[b],
      ).wait()

    for b in range(_NBUF):
      shift_chunk(b)
      start(b, b)

    def group_body(g, carry):
      for b in range(_NBUF):
        chunk = g * _NBUF + b
        shift_chunk(chunk + _NBUF)
        drain(chunk, b)
        pltpu.sync_copy(
            bufs[b], out_hbm.at[pl.ds(base + chunk * _CHUNK, _CHUNK)]
        )
        start(chunk + _NBUF, b)
      return carry

    lax.fori_loop(0, n_groups - 1, group_body, 0)

    for b in range(_NBUF):
      chunk = (n_groups - 1) * _NBUF + b
      drain(chunk, b)
      pltpu.sync_copy(
          bufs[b], out_hbm.at[pl.ds(base + chunk * _CHUNK, _CHUNK)]
      )

  return gather_kernel


def kernel(data, child_idx, depth):
  n, c = data.shape
  (m,) = child_idx.shape
  return _make_sc_gather(n, c, m)(data, child_idx)


# speedup vs baseline: 1.0081x; 1.0081x over previous
import functools

import jax
import jax.numpy as jnp
from jax import lax
from jax.experimental import pallas as pl
from jax.experimental.pallas import tpu as pltpu
from jax.experimental.pallas import tpu_sc as plsc

_SUB = 64
_BIG = 256
_NSUB = _BIG // _SUB
_NBUF = 3
_LANES = 16


def _make_sc_gather(n, c, m):
  info = plsc.get_sparse_core_info()
  nw = info.num_cores * info.num_subcores
  rows_per_w = m // nw
  n_bigs = rows_per_w // _BIG
  assert m == nw * rows_per_w and rows_per_w == n_bigs * _BIG

  mesh = plsc.VectorSubcoreMesh(core_axis_name="c", subcore_axis_name="s")

  @functools.partial(
      pl.kernel,
      out_type=jax.ShapeDtypeStruct((m, c), jnp.float32),
      mesh=mesh,
      scratch_types=(
          [pltpu.VMEM((rows_per_w,), jnp.int32)]
          + [pltpu.VMEM((_BIG, c), jnp.float32) for _ in range(_NBUF)]
          + [pltpu.SemaphoreType.DMA for _ in range(_NBUF)]
      ),
  )
  def gather_kernel(data_hbm, idx_hbm, out_hbm, idx_v, *bufs_sems):
    bufs = bufs_sems[:_NBUF]
    sems = bufs_sems[_NBUF:]
    wid = lax.axis_index("s") * info.num_cores + lax.axis_index("c")
    base = wid * rows_per_w

    pltpu.sync_copy(idx_hbm.at[pl.ds(base, rows_per_w)], idx_v)

    def shift_big(k):
      for i in range(_BIG // _LANES):
        sl = pl.ds(k * _BIG + i * _LANES, _LANES)
        idx_v[sl] = lax.shift_right_logical(idx_v[sl], 3)

    def start(k, b):
      for s in range(_NSUB):
        pltpu.async_copy(
            data_hbm.at[idx_v.at[pl.ds(k * _BIG + s * _SUB, _SUB)]],
            bufs[b].at[pl.ds(s * _SUB, _SUB)],
            sems[b],
        )

    def drain(k, b):
      for s in range(_NSUB):
        pltpu.make_async_copy(
            data_hbm.at[idx_v.at[pl.ds(k * _BIG + s * _SUB, _SUB)]],
            bufs[b].at[pl.ds(s * _SUB, _SUB)],
            sems[b],
        ).wait()

    for b in range(_NBUF):
      shift_big(b)
      start(b, b)

    def group_body(g, carry):
      for j in range(_NBUF):
        k = g * _NBUF + j
        drain(k, j)
        pltpu.sync_copy(
            bufs[j], out_hbm.at[pl.ds(base + k * _BIG, _BIG)]
        )

        @pl.when(k + _NBUF < n_bigs)
        def _():
          shift_big(k + _NBUF)
          start(k + _NBUF, j)

      return carry

    n_groups = n_bigs // _NBUF
    lax.fori_loop(0, n_groups, group_body, 0)

    for k in range(n_groups * _NBUF, n_bigs):
      j = k % _NBUF
      drain(k, j)
      pltpu.sync_copy(bufs[j], out_hbm.at[pl.ds(base + k * _BIG, _BIG)])

  return gather_kernel


def kernel(data, child_idx, depth):
  n, c = data.shape
  (m,) = child_idx.shape
  return _make_sc_gather(n, c, m)(data, child_idx)
